# baseline (speedup 1.0000x reference)
import jax
import jax.numpy as jnp
from jax.experimental import pallas as pl
from jax.experimental.pallas import tpu as pltpu

_B, _N, _D = 16, 128, 512
_RH, _RO = 256, 64
_GO = 512
_E = 128
_G = 2


def _body(x_ref, scal_ref, i0_ref, i1_ref,
          sW0_ref, sb0_ref, sW1_ref, sb1_ref,
          oW0_ref, ob0_ref, oW1_ref, ob1_ref,
          gW_ref, gb_ref, out_ref):
    f32, bf16 = jnp.float32, jnp.bfloat16
    xf = x_ref[...].reshape(_G * _N, _D)

    hs = jnp.maximum(jnp.dot(xf, sW0_ref[...], preferred_element_type=f32)
                     + sb0_ref[...], 0.0)
    phi_f = jnp.dot(hs, sW1_ref[...], preferred_element_type=f32) + sb1_ref[...]
    ho = jnp.maximum(jnp.dot(xf, oW0_ref[...], preferred_element_type=f32)
                     + ob0_ref[...], 0.0)
    psi_f = jnp.dot(ho, oW1_ref[...], preferred_element_type=f32) + ob1_ref[...]

    rowi = jax.lax.broadcasted_iota(jnp.int32, (_N, _N), 0)
    colj = jax.lax.broadcasted_iota(jnp.int32, (_N, _N), 1)
    upper = (rowi <= colj).astype(bf16)
    lstrict = (colj < rowi).astype(bf16)
    eidx = jax.lax.broadcasted_iota(jnp.int32, (1, _E), 1)
    ii = jax.lax.broadcasted_iota(jnp.int32, (_N, _E), 0)
    ones = jnp.ones((_N, 1), f32)

    aggs = []
    for g in range(_G):
        ne = scal_ref[g, 0, 0]
        no = scal_ref[g, 0, 1]
        phi = phi_f[g * _N:(g + 1) * _N]
        psi = psi_f[g * _N:(g + 1) * _N]
        logit = jax.lax.dot_general(phi, psi, (((1,), (1,)), ((), ())),
                                    preferred_element_type=f32)
        rel = logit > 0.0
        rel = rel & ~((rowi >= no) & (colj >= no))

        i0 = i0_ref[g]
        i1 = i1_ref[g]
        valid = (eidx < ne).astype(bf16)
        a1 = (ii == i0).astype(bf16) * valid
        b1 = (ii == i1).astype(bf16)
        a2 = (ii == i1).astype(bf16) * valid
        b2 = (ii == i0).astype(bf16)
        cnt = (jax.lax.dot_general(a1, b1, (((1,), (1,)), ((), ())),
                                   preferred_element_type=f32)
               + jax.lax.dot_general(a2, b2, (((1,), (1,)), ((), ())),
                                     preferred_element_type=f32))
        rel = rel | (cnt > 0.5)
        relb = rel.astype(bf16)

        c_row = jnp.dot(relb, upper, preferred_element_type=f32)
        rtot = c_row[:, _N - 1:_N].astype(bf16)
        pref = jnp.dot(lstrict, rtot, preferred_element_type=f32)
        c = c_row + pref
        keep = relb.astype(f32) * (c <= float(_E)).astype(f32)

        x_g = xf[g * _N:(g + 1) * _N]
        agg = jax.lax.dot_general(keep, x_g, (((0,), (0,)), ((), ())),
                                  preferred_element_type=f32)
        deg = jax.lax.dot_general(keep, ones, (((0,), (0,)), ((), ())),
                                  preferred_element_type=f32)
        aggs.append(agg / jnp.maximum(deg, 1.0))

    y = xf + jnp.concatenate(aggs, axis=0)
    out = jnp.maximum(jnp.dot(y, gW_ref[...], preferred_element_type=f32)
                      + gb_ref[...], 0.0)
    out_ref[...] = out.reshape(_G, _N, _GO)


def kernel(concatenated_node_features, num_obj, num_edges, object_pairs,
           subj_W0, subj_b0, subj_W1, subj_b1,
           obj_W0, obj_b0, obj_W1, obj_b1,
           gcn_W, gcn_b):
    x = concatenated_node_features
    scal = jnp.concatenate(
        [num_edges.reshape(_B, 1).astype(jnp.int32),
         jnp.full((_B, 1), num_obj, dtype=jnp.int32)], axis=1).reshape(_B, 1, 2)
    i0 = object_pairs[:, :, 0].astype(jnp.int32).reshape(_B, 1, _E)
    i1 = object_pairs[:, :, 1].astype(jnp.int32).reshape(_B, 1, _E)

    steps = _B // _G
    const2 = lambda shape: pl.BlockSpec(shape, lambda b: (0, 0))
    out = pl.pallas_call(
        _body,
        grid=(steps,),
        in_specs=[
            pl.BlockSpec((_G, _N, _D), lambda b: (b, 0, 0)),
            pl.BlockSpec((_G, 1, 2), lambda b: (b, 0, 0),
                         memory_space=pltpu.MemorySpace.SMEM),
            pl.BlockSpec((_G, 1, _E), lambda b: (b, 0, 0)),
            pl.BlockSpec((_G, 1, _E), lambda b: (b, 0, 0)),
            const2((_D, _RH)), const2((1, _RH)),
            const2((_RH, _RO)), const2((1, _RO)),
            const2((_D, _RH)), const2((1, _RH)),
            const2((_RH, _RO)), const2((1, _RO)),
            const2((_D, _GO)), const2((1, _GO)),
        ],
        out_specs=pl.BlockSpec((_G, _N, _GO), lambda b: (b, 0, 0)),
        out_shape=jax.ShapeDtypeStruct((_B, _N, _GO), jnp.float32),
        compiler_params=pltpu.CompilerParams(
            dimension_semantics=("parallel",)),
    )(x, scal, i0, i1,
      subj_W0, subj_b0.reshape(1, _RH), subj_W1, subj_b1.reshape(1, _RO),
      obj_W0, obj_b0.reshape(1, _RH), obj_W1, obj_b1.reshape(1, _RO),
      gcn_W, gcn_b.reshape(1, _GO))
    return out.reshape(_B * _N, _GO)


# speedup vs baseline: 1.5175x; 1.5175x over previous
import jax
import jax.numpy as jnp
from jax.experimental import pallas as pl
from jax.experimental.pallas import tpu as pltpu

_B, _N, _D = 16, 128, 512
_RH, _RO = 256, 64
_GO = 512
_E = 128
_G = 4


def _body(x_ref, scal_ref, i0_ref, i1_ref,
          sW0_ref, sb0_ref, sW1_ref, sb1_ref,
          oW0_ref, ob0_ref, oW1_ref, ob1_ref,
          gW_ref, gb_ref, out_ref):
    f32, bf16 = jnp.float32, jnp.bfloat16
    xf = x_ref[...].reshape(_G * _N, _D)

    hs = jnp.maximum(jnp.dot(xf, sW0_ref[...], preferred_element_type=f32)
                     + sb0_ref[...], 0.0)
    phi_f = jnp.dot(hs, sW1_ref[...], preferred_element_type=f32) + sb1_ref[...]
    ho = jnp.maximum(jnp.dot(xf, oW0_ref[...], preferred_element_type=f32)
                     + ob0_ref[...], 0.0)
    psi_f = jnp.dot(ho, oW1_ref[...], preferred_element_type=f32) + ob1_ref[...]

    rowi = jax.lax.broadcasted_iota(jnp.int32, (_N, _N), 0)
    colj = jax.lax.broadcasted_iota(jnp.int32, (_N, _N), 1)
    upper = (rowi <= colj).astype(bf16)
    lstrict = (colj < rowi).astype(bf16)
    eidx = jax.lax.broadcasted_iota(jnp.int32, (1, _E), 1)
    ii = jax.lax.broadcasted_iota(jnp.int32, (_N, _E), 0)
    ones = jnp.ones((_N, 1), f32)

    ones_b = jnp.ones((_N, 1), bf16)

    logits, cnts, relbs, c_rows, rtots, prefs, keeps = ({} for _ in range(7))
    for g in range(_G):
        phi = phi_f[g * _N:(g + 1) * _N]
        psi = psi_f[g * _N:(g + 1) * _N]
        logits[g] = jax.lax.dot_general(phi, psi, (((1,), (1,)), ((), ())),
                                        preferred_element_type=f32)
    for g in range(_G):
        ne = scal_ref[g, 0, 0]
        valid = (eidx < ne).astype(bf16)
        a1 = (ii == i0_ref[g]).astype(bf16) * valid
        b1 = (ii == i1_ref[g]).astype(bf16)
        a2 = (ii == i1_ref[g]).astype(bf16) * valid
        b2 = (ii == i0_ref[g]).astype(bf16)
        cnts[g] = (jax.lax.dot_general(a1, b1, (((1,), (1,)), ((), ())),
                                       preferred_element_type=f32)
                   + jax.lax.dot_general(a2, b2, (((1,), (1,)), ((), ())),
                                         preferred_element_type=f32))
    for g in range(_G):
        no = scal_ref[g, 0, 1]
        rel = logits[g] > 0.0
        rel = rel & ~((rowi >= no) & (colj >= no))
        relbs[g] = (rel | (cnts[g] > 0.5)).astype(bf16)
    for g in range(_G):
        c_rows[g] = jnp.dot(relbs[g], upper, preferred_element_type=f32)
        rtots[g] = jnp.dot(relbs[g], ones_b,
                           preferred_element_type=f32).astype(bf16)
    for g in range(_G):
        prefs[g] = jnp.dot(lstrict, rtots[g], preferred_element_type=f32)
    for g in range(_G):
        c = c_rows[g] + prefs[g]
        keeps[g] = relbs[g].astype(f32) * (c <= float(_E)).astype(f32)
    aggs = []
    for g in range(_G):
        x_g = xf[g * _N:(g + 1) * _N]
        agg = jax.lax.dot_general(keeps[g], x_g, (((0,), (0,)), ((), ())),
                                  preferred_element_type=f32)
        deg = jax.lax.dot_general(keeps[g], ones, (((0,), (0,)), ((), ())),
                                  preferred_element_type=f32)
        aggs.append(agg / jnp.maximum(deg, 1.0))

    y = xf + jnp.concatenate(aggs, axis=0)
    out = jnp.maximum(jnp.dot(y, gW_ref[...], preferred_element_type=f32)
                      + gb_ref[...], 0.0)
    out_ref[...] = out.reshape(_G, _N, _GO)


def kernel(concatenated_node_features, num_obj, num_edges, object_pairs,
           subj_W0, subj_b0, subj_W1, subj_b1,
           obj_W0, obj_b0, obj_W1, obj_b1,
           gcn_W, gcn_b):
    x = concatenated_node_features
    scal = jnp.concatenate(
        [num_edges.reshape(_B, 1).astype(jnp.int32),
         jnp.full((_B, 1), num_obj, dtype=jnp.int32)], axis=1).reshape(_B, 1, 2)
    i0 = object_pairs[:, :, 0].astype(jnp.int32).reshape(_B, 1, _E)
    i1 = object_pairs[:, :, 1].astype(jnp.int32).reshape(_B, 1, _E)

    steps = _B // _G
    const2 = lambda shape: pl.BlockSpec(shape, lambda b: (0, 0))
    out = pl.pallas_call(
        _body,
        grid=(steps,),
        in_specs=[
            pl.BlockSpec((_G, _N, _D), lambda b: (b, 0, 0)),
            pl.BlockSpec((_G, 1, 2), lambda b: (b, 0, 0),
                         memory_space=pltpu.MemorySpace.SMEM),
            pl.BlockSpec((_G, 1, _E), lambda b: (b, 0, 0)),
            pl.BlockSpec((_G, 1, _E), lambda b: (b, 0, 0)),
            const2((_D, _RH)), const2((1, _RH)),
            const2((_RH, _RO)), const2((1, _RO)),
            const2((_D, _RH)), const2((1, _RH)),
            const2((_RH, _RO)), const2((1, _RO)),
            const2((_D, _GO)), const2((1, _GO)),
        ],
        out_specs=pl.BlockSpec((_G, _N, _GO), lambda b: (b, 0, 0)),
        out_shape=jax.ShapeDtypeStruct((_B, _N, _GO), jnp.float32),
        compiler_params=pltpu.CompilerParams(
            dimension_semantics=("parallel",)),
    )(x, scal, i0, i1,
      subj_W0, subj_b0.reshape(1, _RH), subj_W1, subj_b1.reshape(1, _RO),
      obj_W0, obj_b0.reshape(1, _RH), obj_W1, obj_b1.reshape(1, _RO),
      gcn_W, gcn_b.reshape(1, _GO))
    return out.reshape(_B * _N, _GO)


# speedup vs baseline: 1.6683x; 1.0994x over previous
import jax
import jax.numpy as jnp
from jax.experimental import pallas as pl
from jax.experimental.pallas import tpu as pltpu

_B, _N, _D = 16, 128, 512
_RH, _RO = 256, 64
_GO = 512
_E = 128
_G = 16


def _body(x_ref, scal_ref, i0_ref, i1_ref,
          sW0_ref, sb0_ref, sW1_ref, sb1_ref,
          oW0_ref, ob0_ref, oW1_ref, ob1_ref,
          gW_ref, gb_ref, out_ref):
    f32, bf16 = jnp.float32, jnp.bfloat16
    xf = x_ref[...].reshape(_G * _N, _D)

    hs = jnp.maximum(jnp.dot(xf, sW0_ref[...], preferred_element_type=f32)
                     + sb0_ref[...], 0.0)
    phi_f = jnp.dot(hs, sW1_ref[...], preferred_element_type=f32) + sb1_ref[...]
    ho = jnp.maximum(jnp.dot(xf, oW0_ref[...], preferred_element_type=f32)
                     + ob0_ref[...], 0.0)
    psi_f = jnp.dot(ho, oW1_ref[...], preferred_element_type=f32) + ob1_ref[...]

    rowi = jax.lax.broadcasted_iota(jnp.int32, (_N, _N), 0)
    colj = jax.lax.broadcasted_iota(jnp.int32, (_N, _N), 1)
    upper = (rowi <= colj).astype(bf16)
    lstrict = (colj < rowi).astype(bf16)
    eidx = jax.lax.broadcasted_iota(jnp.int32, (1, _E), 1)
    ii = jax.lax.broadcasted_iota(jnp.int32, (_N, _E), 0)
    ones = jnp.ones((_N, 1), f32)

    ones_b = jnp.ones((_N, 1), bf16)

    logits, cnts, relbs, c_rows, rtots, prefs, keeps = ({} for _ in range(7))
    for g in range(_G):
        phi = phi_f[g * _N:(g + 1) * _N]
        psi = psi_f[g * _N:(g + 1) * _N]
        logits[g] = jax.lax.dot_general(phi, psi, (((1,), (1,)), ((), ())),
                                        preferred_element_type=f32)
    for g in range(_G):
        ne = scal_ref[g, 0, 0]
        valid = (eidx < ne).astype(bf16)
        a1 = (ii == i0_ref[g]).astype(bf16) * valid
        b1 = (ii == i1_ref[g]).astype(bf16)
        a2 = (ii == i1_ref[g]).astype(bf16) * valid
        b2 = (ii == i0_ref[g]).astype(bf16)
        cnts[g] = (jax.lax.dot_general(a1, b1, (((1,), (1,)), ((), ())),
                                       preferred_element_type=f32)
                   + jax.lax.dot_general(a2, b2, (((1,), (1,)), ((), ())),
                                         preferred_element_type=f32))
    for g in range(_G):
        no = scal_ref[g, 0, 1]
        rel = logits[g] > 0.0
        rel = rel & ~((rowi >= no) & (colj >= no))
        relbs[g] = (rel | (cnts[g] > 0.5)).astype(bf16)
    for g in range(_G):
        c_rows[g] = jnp.dot(relbs[g], upper, preferred_element_type=f32)
        rtots[g] = jnp.dot(relbs[g], ones_b,
                           preferred_element_type=f32).astype(bf16)
    for g in range(_G):
        prefs[g] = jnp.dot(lstrict, rtots[g], preferred_element_type=f32)
    for g in range(_G):
        c = c_rows[g] + prefs[g]
        keeps[g] = relbs[g].astype(f32) * (c <= float(_E)).astype(f32)
    aggs = []
    for g in range(_G):
        x_g = xf[g * _N:(g + 1) * _N]
        agg = jax.lax.dot_general(keeps[g], x_g, (((0,), (0,)), ((), ())),
                                  preferred_element_type=f32)
        deg = jax.lax.dot_general(keeps[g], ones, (((0,), (0,)), ((), ())),
                                  preferred_element_type=f32)
        aggs.append(agg / jnp.maximum(deg, 1.0))

    y = xf + jnp.concatenate(aggs, axis=0)
    out = jnp.maximum(jnp.dot(y, gW_ref[...], preferred_element_type=f32)
                      + gb_ref[...], 0.0)
    out_ref[...] = out.reshape(_G, _N, _GO)


def kernel(concatenated_node_features, num_obj, num_edges, object_pairs,
           subj_W0, subj_b0, subj_W1, subj_b1,
           obj_W0, obj_b0, obj_W1, obj_b1,
           gcn_W, gcn_b):
    x = concatenated_node_features
    scal = jnp.concatenate(
        [num_edges.reshape(_B, 1).astype(jnp.int32),
         jnp.full((_B, 1), num_obj, dtype=jnp.int32)], axis=1).reshape(_B, 1, 2)
    i0 = object_pairs[:, :, 0].astype(jnp.int32).reshape(_B, 1, _E)
    i1 = object_pairs[:, :, 1].astype(jnp.int32).reshape(_B, 1, _E)

    steps = _B // _G
    const2 = lambda shape: pl.BlockSpec(shape, lambda b: (0, 0))
    out = pl.pallas_call(
        _body,
        grid=(steps,),
        in_specs=[
            pl.BlockSpec((_G, _N, _D), lambda b: (b, 0, 0)),
            pl.BlockSpec((_G, 1, 2), lambda b: (b, 0, 0),
                         memory_space=pltpu.MemorySpace.SMEM),
            pl.BlockSpec((_G, 1, _E), lambda b: (b, 0, 0)),
            pl.BlockSpec((_G, 1, _E), lambda b: (b, 0, 0)),
            const2((_D, _RH)), const2((1, _RH)),
            const2((_RH, _RO)), const2((1, _RO)),
            const2((_D, _RH)), const2((1, _RH)),
            const2((_RH, _RO)), const2((1, _RO)),
            const2((_D, _GO)), const2((1, _GO)),
        ],
        out_specs=pl.BlockSpec((_G, _N, _GO), lambda b: (b, 0, 0)),
        out_shape=jax.ShapeDtypeStruct((_B, _N, _GO), jnp.float32),
        compiler_params=pltpu.CompilerParams(
            dimension_semantics=("parallel",)),
    )(x, scal, i0, i1,
      subj_W0, subj_b0.reshape(1, _RH), subj_W1, subj_b1.reshape(1, _RO),
      obj_W0, obj_b0.reshape(1, _RH), obj_W1, obj_b1.reshape(1, _RO),
      gcn_W, gcn_b.reshape(1, _GO))
    return out.reshape(_B * _N, _GO)


# speedup vs baseline: 1.6822x; 1.0084x over previous
import jax
import jax.numpy as jnp
from jax.experimental import pallas as pl
from jax.experimental.pallas import tpu as pltpu

_B, _N, _D = 16, 128, 512
_RH, _RO = 256, 64
_GO = 512
_E = 128
_G = 8


def _body(x_ref, scal_ref, i0_ref, i1_ref,
          sW0_ref, sb0_ref, sW1_ref, sb1_ref,
          oW0_ref, ob0_ref, oW1_ref, ob1_ref,
          gW_ref, gb_ref, out_ref):
    f32, bf16 = jnp.float32, jnp.bfloat16
    xf = x_ref[...].reshape(_G * _N, _D)

    hs = jnp.maximum(jnp.dot(xf, sW0_ref[...], preferred_element_type=f32)
                     + sb0_ref[...], 0.0)
    phi_f = jnp.dot(hs, sW1_ref[...], preferred_element_type=f32) + sb1_ref[...]
    ho = jnp.maximum(jnp.dot(xf, oW0_ref[...], preferred_element_type=f32)
                     + ob0_ref[...], 0.0)
    psi_f = jnp.dot(ho, oW1_ref[...], preferred_element_type=f32) + ob1_ref[...]

    rowi = jax.lax.broadcasted_iota(jnp.int32, (_N, _N), 0)
    colj = jax.lax.broadcasted_iota(jnp.int32, (_N, _N), 1)
    upper = (rowi <= colj).astype(bf16)
    lstrict = (colj < rowi).astype(bf16)
    eidx = jax.lax.broadcasted_iota(jnp.int32, (1, _E), 1)
    ii = jax.lax.broadcasted_iota(jnp.int32, (_N, _E), 0)
    ones = jnp.ones((_N, 1), f32)

    ones_b = jnp.ones((_N, 1), bf16)

    logits, cnts, relbs, c_rows, rtots, prefs, keeps = ({} for _ in range(7))
    for g in range(_G):
        phi = phi_f[g * _N:(g + 1) * _N]
        psi = psi_f[g * _N:(g + 1) * _N]
        logits[g] = jax.lax.dot_general(phi, psi, (((1,), (1,)), ((), ())),
                                        preferred_element_type=f32)
    for g in range(_G):
        ne = scal_ref[g, 0, 0]
        valid = (eidx < ne).astype(bf16)
        a1 = (ii == i0_ref[g]).astype(bf16) * valid
        b1 = (ii == i1_ref[g]).astype(bf16)
        a2 = (ii == i1_ref[g]).astype(bf16) * valid
        b2 = (ii == i0_ref[g]).astype(bf16)
        cnts[g] = (jax.lax.dot_general(a1, b1, (((1,), (1,)), ((), ())),
                                       preferred_element_type=f32)
                   + jax.lax.dot_general(a2, b2, (((1,), (1,)), ((), ())),
                                         preferred_element_type=f32))
    for g in range(_G):
        no = scal_ref[g, 0, 1]
        rel = logits[g] > 0.0
        rel = rel & ~((rowi >= no) & (colj >= no))
        relbs[g] = (rel | (cnts[g] > 0.5)).astype(bf16)
    for g in range(_G):
        c_rows[g] = jnp.dot(relbs[g], upper, preferred_element_type=f32)
        rtots[g] = jnp.dot(relbs[g], ones_b,
                           preferred_element_type=f32).astype(bf16)
    for g in range(_G):
        prefs[g] = jnp.dot(lstrict, rtots[g], preferred_element_type=f32)
    for g in range(_G):
        c = c_rows[g] + prefs[g]
        keeps[g] = relbs[g].astype(f32) * (c <= float(_E)).astype(f32)
    aggs = []
    for g in range(_G):
        x_g = xf[g * _N:(g + 1) * _N]
        agg = jax.lax.dot_general(keeps[g], x_g, (((0,), (0,)), ((), ())),
                                  preferred_element_type=f32)
        deg = jax.lax.dot_general(keeps[g], ones, (((0,), (0,)), ((), ())),
                                  preferred_element_type=f32)
        aggs.append(agg / jnp.maximum(deg, 1.0))

    y = xf + jnp.concatenate(aggs, axis=0)
    out = jnp.maximum(jnp.dot(y, gW_ref[...], preferred_element_type=f32)
                      + gb_ref[...], 0.0)
    out_ref[...] = out.reshape(_G, _N, _GO)


def kernel(concatenated_node_features, num_obj, num_edges, object_pairs,
           subj_W0, subj_b0, subj_W1, subj_b1,
           obj_W0, obj_b0, obj_W1, obj_b1,
           gcn_W, gcn_b):
    x = concatenated_node_features
    scal = jnp.concatenate(
        [num_edges.reshape(_B, 1).astype(jnp.int32),
         jnp.full((_B, 1), num_obj, dtype=jnp.int32)], axis=1).reshape(_B, 1, 2)
    i0 = object_pairs[:, :, 0].astype(jnp.int32).reshape(_B, 1, _E)
    i1 = object_pairs[:, :, 1].astype(jnp.int32).reshape(_B, 1, _E)

    steps = _B // _G
    const2 = lambda shape: pl.BlockSpec(shape, lambda b: (0, 0))
    out = pl.pallas_call(
        _body,
        grid=(steps,),
        in_specs=[
            pl.BlockSpec((_G, _N, _D), lambda b: (b, 0, 0)),
            pl.BlockSpec((_G, 1, 2), lambda b: (b, 0, 0),
                         memory_space=pltpu.MemorySpace.SMEM),
            pl.BlockSpec((_G, 1, _E), lambda b: (b, 0, 0)),
            pl.BlockSpec((_G, 1, _E), lambda b: (b, 0, 0)),
            const2((_D, _RH)), const2((1, _RH)),
            const2((_RH, _RO)), const2((1, _RO)),
            const2((_D, _RH)), const2((1, _RH)),
            const2((_RH, _RO)), const2((1, _RO)),
            const2((_D, _GO)), const2((1, _GO)),
        ],
        out_specs=pl.BlockSpec((_G, _N, _GO), lambda b: (b, 0, 0)),
        out_shape=jax.ShapeDtypeStruct((_B, _N, _GO), jnp.float32),
        compiler_params=pltpu.CompilerParams(
            dimension_semantics=("parallel",)),
    )(x, scal, i0, i1,
      subj_W0, subj_b0.reshape(1, _RH), subj_W1, subj_b1.reshape(1, _RO),
      obj_W0, obj_b0.reshape(1, _RH), obj_W1, obj_b1.reshape(1, _RO),
      gcn_W, gcn_b.reshape(1, _GO))
    return out.reshape(_B * _N, _GO)


# speedup vs baseline: 1.6931x; 1.0065x over previous
import jax
import jax.numpy as jnp
from jax.experimental import pallas as pl
from jax.experimental.pallas import tpu as pltpu

_B, _N, _D = 16, 128, 512
_RH, _RO = 256, 64
_GO = 512
_E = 128
_G = 8


def _body(x_ref, scal_ref, i0_ref, i1_ref,
          sW0_ref, sb0_ref, sW1_ref, sb1_ref,
          oW0_ref, ob0_ref, oW1_ref, ob1_ref,
          gW_ref, gb_ref, out_ref):
    f32, bf16 = jnp.float32, jnp.bfloat16
    xf = x_ref[...].reshape(_G * _N, _D)

    hs = jnp.maximum(jnp.dot(xf, sW0_ref[...], preferred_element_type=f32)
                     + sb0_ref[...], 0.0)
    phi_f = jnp.dot(hs, sW1_ref[...], preferred_element_type=f32) + sb1_ref[...]
    ho = jnp.maximum(jnp.dot(xf, oW0_ref[...], preferred_element_type=f32)
                     + ob0_ref[...], 0.0)
    psi_f = jnp.dot(ho, oW1_ref[...], preferred_element_type=f32) + ob1_ref[...]

    rowi = jax.lax.broadcasted_iota(jnp.int32, (_N, _N), 0)
    colj = jax.lax.broadcasted_iota(jnp.int32, (_N, _N), 1)
    upper = (rowi <= colj).astype(bf16)
    lstrict = (colj < rowi).astype(bf16)
    eidx = jax.lax.broadcasted_iota(jnp.int32, (1, _E), 1)
    ii = jax.lax.broadcasted_iota(jnp.int32, (_N, _E), 0)
    ones = jnp.ones((_N, 1), f32)

    ones_b = jnp.ones((_N, 1), bf16)

    logits, cnts, relbs, c_rows, rtots, prefs, keeps = ({} for _ in range(7))
    for g in range(_G):
        phi = phi_f[g * _N:(g + 1) * _N]
        psi = psi_f[g * _N:(g + 1) * _N]
        logits[g] = jax.lax.dot_general(phi, psi, (((1,), (1,)), ((), ())),
                                        preferred_element_type=f32)
    for g in range(_G):
        ne = scal_ref[g, 0, 0]
        valid = (eidx < ne).astype(bf16)
        a1 = (ii == i0_ref[g]).astype(bf16) * valid
        b1 = (ii == i1_ref[g]).astype(bf16)
        cnts[g] = jax.lax.dot_general(a1, b1, (((1,), (1,)), ((), ())),
                                      preferred_element_type=f32)
    no = scal_ref[0, 0, 1]
    tail = (rowi >= no) & (colj >= no)
    for g in range(_G):
        rel = (logits[g] > 0.0) & ~tail
        pair = (cnts[g] + cnts[g].T) > 0.5
        relbs[g] = (rel | pair).astype(bf16)
    for g in range(_G):
        c_rows[g] = jnp.dot(relbs[g], upper, preferred_element_type=f32)
        rtots[g] = jnp.dot(relbs[g], ones_b,
                           preferred_element_type=f32).astype(bf16)
    for g in range(_G):
        prefs[g] = jnp.dot(lstrict, rtots[g], preferred_element_type=f32)
    for g in range(_G):
        c = c_rows[g] + prefs[g]
        keeps[g] = relbs[g].astype(f32) * (c <= float(_E)).astype(f32)
    aggs = []
    for g in range(_G):
        x_g = xf[g * _N:(g + 1) * _N]
        agg = jax.lax.dot_general(keeps[g], x_g, (((0,), (0,)), ((), ())),
                                  preferred_element_type=f32)
        deg = jax.lax.dot_general(keeps[g], ones, (((0,), (0,)), ((), ())),
                                  preferred_element_type=f32)
        aggs.append(agg / jnp.maximum(deg, 1.0))

    y = xf + jnp.concatenate(aggs, axis=0)
    out = jnp.maximum(jnp.dot(y, gW_ref[...], preferred_element_type=f32)
                      + gb_ref[...], 0.0)
    out_ref[...] = out.reshape(_G, _N, _GO)


def kernel(concatenated_node_features, num_obj, num_edges, object_pairs,
           subj_W0, subj_b0, subj_W1, subj_b1,
           obj_W0, obj_b0, obj_W1, obj_b1,
           gcn_W, gcn_b):
    x = concatenated_node_features
    scal = jnp.concatenate(
        [num_edges.reshape(_B, 1).astype(jnp.int32),
         jnp.full((_B, 1), num_obj, dtype=jnp.int32)], axis=1).reshape(_B, 1, 2)
    i0 = object_pairs[:, :, 0].astype(jnp.int32).reshape(_B, 1, _E)
    i1 = object_pairs[:, :, 1].astype(jnp.int32).reshape(_B, 1, _E)

    steps = _B // _G
    const2 = lambda shape: pl.BlockSpec(shape, lambda b: (0, 0))
    out = pl.pallas_call(
        _body,
        grid=(steps,),
        in_specs=[
            pl.BlockSpec((_G, _N, _D), lambda b: (b, 0, 0)),
            pl.BlockSpec((_G, 1, 2), lambda b: (b, 0, 0),
                         memory_space=pltpu.MemorySpace.SMEM),
            pl.BlockSpec((_G, 1, _E), lambda b: (b, 0, 0)),
            pl.BlockSpec((_G, 1, _E), lambda b: (b, 0, 0)),
            const2((_D, _RH)), const2((1, _RH)),
            const2((_RH, _RO)), const2((1, _RO)),
            const2((_D, _RH)), const2((1, _RH)),
            const2((_RH, _RO)), const2((1, _RO)),
            const2((_D, _GO)), const2((1, _GO)),
        ],
        out_specs=pl.BlockSpec((_G, _N, _GO), lambda b: (b, 0, 0)),
        out_shape=jax.ShapeDtypeStruct((_B, _N, _GO), jnp.float32),
        compiler_params=pltpu.CompilerParams(
            dimension_semantics=("parallel",)),
    )(x, scal, i0, i1,
      subj_W0, subj_b0.reshape(1, _RH), subj_W1, subj_b1.reshape(1, _RO),
      obj_W0, obj_b0.reshape(1, _RH), obj_W1, obj_b1.reshape(1, _RO),
      gcn_W, gcn_b.reshape(1, _GO))
    return out.reshape(_B * _N, _GO)
